# baseline (speedup 1.0000x reference)
import functools

import jax
import jax.numpy as jnp
from jax import lax
from jax.experimental import pallas as pl
from jax.experimental.pallas import tpu as pltpu
from jax.experimental.pallas import tpu_sc as plsc

B, N, C, H = 4, 10000, 128, 128
E = 320000
NSUB = 16
NCORE = 2
EPT = E // NSUB
S = 128
NCHK = 160
EPT_PAD = NCHK * S
SG = 32
EBLK = 16 * SG
NBLKC = 20480 // EBLK
EPTC = EPT_PAD
NCHKH = EPTC // SG
NH = N // 2
NA_PAD = 5120
DUMPL = NH
RPTA = NA_PAD // NSUB
N_PAD = 10240
DUMP = N
RPT = N_PAD // NSUB
BLK = 1000
NBH = NH // BLK

_mesh = plsc.VectorSubcoreMesh(core_axis_name="c", subcore_axis_name="s")


def _make_prop(npass):
    npc = npass // NCORE

    @functools.partial(
        pl.kernel,
        out_type=jax.ShapeDtypeStruct((npass, NA_PAD, 2, 128), jnp.float32),
        mesh=_mesh,
        scratch_types=[
            pltpu.VMEM((2, 16, SG), jnp.int32),
            pltpu.VMEM((2, 16, SG), jnp.int32),
            pltpu.VMEM((16,), jnp.int32),
            [pltpu.VMEM((SG, 2, 128), jnp.float32) for _ in range(4)],
            pltpu.VMEM_SHARED((NA_PAD, 2, 128), jnp.float32),
            pltpu.SemaphoreType.DMA,
            [pltpu.SemaphoreType.DMA for _ in range(4)],
            [pltpu.SemaphoreType.DMA for _ in range(4)],
        ],
    )
    def prop(ytbl, rowp, colp, npt, zeros, out, rblk, cblk, npv, gbufs,
             accs, semI, semG, semS):
        ci = lax.axis_index("c")
        s = lax.axis_index("s")
        pltpu.sync_copy(npt.at[s], npv)
        npvec = npv[...]

        for j_pass in range(npc):
            p = ci * npc + j_pass
            g = p // 2
            sig = j_pass % 2
            nblk = npvec[sig]

            def start_gather(slot, t, u):
                pltpu.async_copy(ytbl.at[g].at[rblk.at[slot, t]], gbufs[u],
                                 semG[u])

            def wait_gather(u):
                pltpu.make_async_copy(ytbl.at[g].at[rblk.at[0, 0]],
                                      gbufs[u], semG[u]).wait()

            def start_scatter(slot, t, u):
                pltpu.async_copy(gbufs[u], accs.at[cblk.at[slot, t]],
                                 semS[u], add=True)

            def wait_scatter(u):
                pltpu.make_async_copy(gbufs[u], accs.at[cblk.at[0, 0]],
                                      semS[u]).wait()

            def fetch_block(j, slot):
                pltpu.async_copy(rowp.at[s, sig, j], rblk.at[slot], semI)
                pltpu.async_copy(colp.at[s, sig, j], cblk.at[slot], semI)

            def drain_block():
                pltpu.make_async_copy(rowp.at[s, 0, 0], rblk.at[0],
                                      semI).wait()
                pltpu.make_async_copy(colp.at[s, 0, 0], cblk.at[0],
                                      semI).wait()

            pltpu.sync_copy(zeros, accs.at[pl.ds(s * RPTA, RPTA)])
            plsc.subcore_barrier()

            fetch_block(0, 0)
            drain_block()
            start_gather(0, 0, 0)
            start_gather(0, 1, 1)

            @pl.loop(0, nblk)
            def _(j):
                jslot = lax.rem(j, 2)
                nslot = lax.rem(j + 1, 2)
                more = j < nblk - 1

                for t in range(16):
                    u = t % 4
                    v = (t + 2) % 4
                    wait_gather(u)
                    start_scatter(jslot, t, u)
                    if t == 1:
                        @pl.when(more)
                        def _():
                            fetch_block(j + 1, nslot)
                    if t < 14:
                        @pl.when(j > 0)
                        def _():
                            wait_scatter(v)
                            start_gather(jslot, t + 2, v)

                        @pl.when(j == 0)
                        def _():
                            if t < 2:
                                start_gather(jslot, t + 2, v)
                            else:
                                wait_scatter(v)
                                start_gather(jslot, t + 2, v)
                    else:
                        if t == 14:
                            @pl.when(more)
                            def _():
                                drain_block()

                        @pl.when(more)
                        def _():
                            wait_scatter(v)
                            start_gather(nslot, t - 14, v)

            for u in range(4):
                wait_scatter(u)

            plsc.subcore_barrier()
            pltpu.sync_copy(accs.at[pl.ds(s * RPTA, RPTA)],
                            out.at[p, pl.ds(s * RPTA, RPTA)])
            plsc.subcore_barrier()

    return prop


@functools.partial(
    pl.kernel,
    out_type=jax.ShapeDtypeStruct((NCORE, N_PAD, 128), jnp.float32),
    mesh=_mesh,
    scratch_types=[
        pltpu.VMEM((NCHK, S), jnp.int32),
        pltpu.VMEM((S, 128), jnp.float32),
        pltpu.VMEM_SHARED((N_PAD, 128), jnp.float32),
    ],
)
def _deg_kernel(colt, ones, zeros, out, col_v, ones_v, accs):
    ci = lax.axis_index("c")
    s = lax.axis_index("s")
    pltpu.sync_copy(colt.at[s], col_v)
    pltpu.sync_copy(ones, ones_v)
    pltpu.sync_copy(zeros, accs.at[pl.ds(s * RPT, RPT)])
    plsc.subcore_barrier()

    @pl.loop(0, NCHK // NCORE)
    def _(k):
        pltpu.sync_copy(ones_v, accs.at[col_v.at[ci * (NCHK // NCORE) + k]],
                        add=True)

    plsc.subcore_barrier()
    pltpu.sync_copy(accs.at[pl.ds(s * RPT, RPT)],
                    out.at[ci, pl.ds(s * RPT, RPT)])


@functools.partial(
    pl.kernel,
    out_type=[
        jax.ShapeDtypeStruct((NSUB, 2, EPTC), jnp.int32),
        jax.ShapeDtypeStruct((NSUB, 2, EPTC), jnp.int32),
        jax.ShapeDtypeStruct((NSUB, 16), jnp.int32),
    ],
    mesh=_mesh,
    scratch_types=[
        pltpu.VMEM((NCHK, S), jnp.int32),
        pltpu.VMEM((NCHK, S), jnp.int32),
        [pltpu.VMEM((EPTC + 512,), jnp.int32) for _ in range(2)],
        [pltpu.VMEM((EPTC + 512,), jnp.int32) for _ in range(2)],
        pltpu.VMEM((16,), jnp.int32),
    ],
    compiler_params=pltpu.CompilerParams(needs_layout_passes=False),
)
def _part_kernel(rowt, colt, rows_out, cols_out, npt_out, row_v, col_v,
                 prow, pcol, npv):
    ci = lax.axis_index("c")
    s = lax.axis_index("s")
    pltpu.sync_copy(rowt.at[s], row_v)
    pltpu.sync_copy(colt.at[s], col_v)

    def body(i, offs):
        off0, off1 = offs
        r = i // (S // 16)
        c = lax.rem(i, S // 16)
        rv = row_v[r, pl.ds(c * 16, 16)]
        cv = col_v[r, pl.ds(c * 16, 16)]
        m1 = cv >= NH
        m0 = jnp.logical_not(m1)
        n1 = plsc.all_reduce_population_count(m1)[0]
        plsc.store_compressed(prow[0].at[pl.ds(off0, 16)], rv, mask=m0)
        plsc.store_compressed(pcol[0].at[pl.ds(off0, 16)], cv, mask=m0)
        plsc.store_compressed(prow[1].at[pl.ds(off1, 16)], rv, mask=m1)
        plsc.store_compressed(pcol[1].at[pl.ds(off1, 16)], cv - NH, mask=m1)
        return off0 + (16 - n1), off1 + n1

    zero = jnp.zeros((), jnp.int32)
    cnt0, cnt1 = pl.loop(0, EPT // 16, init_carry=(zero, zero))(body)

    dumpv = jnp.full((16,), DUMPL, jnp.int32)
    zerov = jnp.zeros((16,), jnp.int32)
    for half, cnt in ((0, cnt0), (1, cnt1)):
        for k in range(32):
            prow[half][pl.ds(cnt + 16 * k, 16)] = zerov
            pcol[half][pl.ds(cnt + 16 * k, 16)] = dumpv

    nblk0 = jnp.maximum((cnt0 + EBLK - 1) // EBLK, 1)
    nblk1 = jnp.maximum((cnt1 + EBLK - 1) // EBLK, 1)
    lane = lax.iota(jnp.int32, 16)
    npv[...] = jnp.where(lane == 0, nblk0,
                         jnp.where(lane == 1, nblk1, 0))
    @pl.when(ci == 0)
    def _():
        for half in range(2):
            pltpu.sync_copy(prow[half].at[pl.ds(0, EPTC)],
                            rows_out.at[s, half])
            pltpu.sync_copy(pcol[half].at[pl.ds(0, EPTC)],
                            cols_out.at[s, half])
        pltpu.sync_copy(npv, npt_out.at[s])


def _dinv_of(degp_ref):
    return lax.rsqrt(degp_ref[0] + degp_ref[1] + 1.0)


def _tc_a_body(x_ref, h_ref, w_ref, degp_ref, y1_ref):
    dinv = _dinv_of(degp_ref)
    y = (jnp.dot(x_ref[0], w_ref[0:C], preferred_element_type=jnp.float32)
         + jnp.dot(h_ref[0], w_ref[C:C + H],
                   preferred_element_type=jnp.float32))
    y1_ref[0] = y * jnp.concatenate([dinv, dinv], axis=1)


def _tc_b_body(acc1_ref, y1_ref, degp_ref, x_ref, h_ref, wh_ref, bzr_ref,
               y2_ref, z_ref):
    for b in range(2):
        dinv = _dinv_of(degp_ref)
        z = jax.nn.sigmoid(dinv * (acc1_ref[b, 0, :, 0] + y1_ref[b][:, :H])
                           + bzr_ref[0])
        r = jax.nn.sigmoid(dinv * (acc1_ref[b, 0, :, 1] + y1_ref[b][:, H:])
                           + bzr_ref[1])
        rh = r * h_ref[b]
        y2 = (jnp.dot(x_ref[b], wh_ref[0:C],
                      preferred_element_type=jnp.float32)
              + jnp.dot(rh, wh_ref[C:C + H],
                        preferred_element_type=jnp.float32))
        y2_ref[0, :, b * H:(b + 1) * H] = y2 * dinv
        z_ref[b] = z


def _tc_c_body(acc2_ref, y2_ref, degp_ref, z_ref, h_ref, bh_ref, out_ref):
    for b in range(2):
        dinv = _dinv_of(degp_ref)
        ht = jnp.tanh(
            dinv * (acc2_ref[0, 0, :, b] + y2_ref[0, :, b * H:(b + 1) * H])
            + bh_ref[0])
        z = z_ref[b]
        out_ref[b] = (1.0 - z) * h_ref[b] + z * ht


def kernel(x, h, edge_index, W_zr, b_zr, W_h, b_h):
    row0 = edge_index[0]
    col0 = edge_index[1]
    colseg = col0.reshape(NSUB, EPT)
    row_d = jnp.pad(row0.reshape(NSUB, EPT),
                    ((0, 0), (0, EPT_PAD - EPT))).reshape(NSUB, NCHK, S)
    col_d = jnp.pad(colseg, ((0, 0), (0, EPT_PAD - EPT)),
                    constant_values=DUMP).reshape(NSUB, NCHK, S)

    rows_l, cols_l, npt = _part_kernel(row_d, col_d)
    rowp = rows_l.reshape(NSUB, 2, NBLKC, 16, SG)
    colp = cols_l.reshape(NSUB, 2, NBLKC, 16, SG)
    zeros_a = jnp.zeros((RPTA, 2, 128), jnp.float32)
    zeros_d = jnp.zeros((RPT, 128), jnp.float32)
    ones = jnp.ones((S, 128), jnp.float32)

    degp = _deg_kernel(col_d, ones, zeros_d)

    bnh = pl.BlockSpec((1, BLK, 128), lambda b, i: (b, i, 0))
    degp_b = pl.BlockSpec((NCORE, BLK, 128), lambda b, i: (0, i, 0))
    y1 = pl.pallas_call(
        _tc_a_body,
        grid=(B, N // BLK),
        in_specs=[
            bnh, bnh,
            pl.BlockSpec((C + H, 2 * H), lambda b, i: (0, 0)),
            degp_b,
        ],
        out_specs=pl.BlockSpec((1, BLK, 256), lambda b, i: (b, i, 0)),
        out_shape=jax.ShapeDtypeStruct((B, N, 256), jnp.float32),
    )(x, h, W_zr, degp)

    acc1 = _make_prop(2 * B)(y1.reshape(B, N, 2, 128), rowp, colp, npt,
                             zeros_a)
    acc1 = acc1.reshape(B, 2, NA_PAD, 2, 128)

    bnh2 = pl.BlockSpec((2, BLK, 128), lambda q, i: (q, i, 0))
    degp_b2 = pl.BlockSpec((NCORE, BLK, 128), lambda q, i: (0, i, 0))
    acc_b = pl.BlockSpec((2, 1, BLK, 2, 128),
                         lambda q, i: (q, i // NBH, i % NBH, 0, 0))
    y2, z = pl.pallas_call(
        _tc_b_body,
        grid=(B // 2, N // BLK),
        in_specs=[
            acc_b,
            pl.BlockSpec((2, BLK, 256), lambda q, i: (q, i, 0)),
            degp_b2,
            bnh2, bnh2,
            pl.BlockSpec((C + H, H), lambda q, i: (0, 0)),
            pl.BlockSpec((2, H), lambda q, i: (0, 0)),
        ],
        out_specs=[
            pl.BlockSpec((1, BLK, 256), lambda q, i: (q, i, 0)),
            bnh2,
        ],
        out_shape=[
            jax.ShapeDtypeStruct((B // 2, N, 256), jnp.float32),
            jax.ShapeDtypeStruct((B, N, 128), jnp.float32),
        ],
    )(acc1, y1, degp, x, h, W_h, b_zr.reshape(2, H))

    acc2 = _make_prop(B)(y2.reshape(B // 2, N, 2, 128), rowp, colp, npt,
                         zeros_a)
    acc2 = acc2.reshape(B // 2, 2, NA_PAD, 2, 128)

    out = pl.pallas_call(
        _tc_c_body,
        grid=(B // 2, N // BLK),
        in_specs=[
            pl.BlockSpec((1, 1, BLK, 2, 128),
                         lambda q, i: (q, i // NBH, i % NBH, 0, 0)),
            pl.BlockSpec((1, BLK, 256), lambda q, i: (q, i, 0)),
            degp_b2, bnh2, bnh2,
            pl.BlockSpec((1, H), lambda q, i: (0, 0)),
        ],
        out_specs=bnh2,
        out_shape=jax.ShapeDtypeStruct((B, N, H), jnp.float32),
    )(acc2, y2, degp, z, h, b_h.reshape(1, H))

    return out


# speedup vs baseline: 1.4781x; 1.4781x over previous
import functools

import jax
import jax.numpy as jnp
from jax import lax
from jax.experimental import pallas as pl
from jax.experimental.pallas import tpu as pltpu
from jax.experimental.pallas import tpu_sc as plsc

B, N, C, H = 4, 10000, 128, 128
E = 320000
NSUB = 16
NCORE = 2
EPT = E // NSUB
S = 128
NCHK = 160
EPT_PAD = NCHK * S
SG = 64
EPTC = EPT_PAD
NCHKH = EPTC // SG
NH = N // 2
NA_PAD = 5120
DUMPL = NH
RPTA = NA_PAD // NSUB
N_PAD = 10240
DUMP = N
RPT = N_PAD // NSUB
BLK = 1000
NBH = NH // BLK

_mesh = plsc.VectorSubcoreMesh(core_axis_name="c", subcore_axis_name="s")


def _make_prop(npass):
    npc = npass // NCORE

    @functools.partial(
        pl.kernel,
        out_type=jax.ShapeDtypeStruct((npass, NA_PAD, 2, 128), jnp.float32),
        mesh=_mesh,
        scratch_types=[
            pltpu.VMEM((2, 2, SG), jnp.int32),
            pltpu.VMEM((2, 2, SG), jnp.int32),
            pltpu.VMEM((16,), jnp.int32),
            pltpu.VMEM((SG, 2, 128), jnp.float32),
            pltpu.VMEM((SG, 2, 128), jnp.float32),
            pltpu.VMEM_SHARED((NA_PAD, 2, 128), jnp.float32),
            pltpu.SemaphoreType.DMA,
            pltpu.SemaphoreType.DMA,
            pltpu.SemaphoreType.DMA,
        ],
    )
    def prop(ytbl, rowp, colp, npt, zeros, out, rring, cring, npv, gA, gB,
             accs, semI, semA, semB):
        ci = lax.axis_index("c")
        s = lax.axis_index("s")
        pltpu.sync_copy(npt.at[s], npv)
        npvec = npv[...]

        for j in range(npc):
            p = ci * npc + j
            g = p // 2
            sig = j % 2
            npair = npvec[sig]

            def fetch_idx(pair, slot):
                for u in range(2):
                    pltpu.async_copy(rowp.at[s, sig, 2 * pair + u],
                                     rring.at[slot, u], semI)
                    pltpu.async_copy(colp.at[s, sig, 2 * pair + u],
                                     cring.at[slot, u], semI)

            def drain_idx():
                for u in range(2):
                    pltpu.make_async_copy(rowp.at[s, 0, 0], rring.at[0, u],
                                          semI).wait()
                    pltpu.make_async_copy(colp.at[s, 0, 0], cring.at[0, u],
                                          semI).wait()

            pltpu.sync_copy(zeros, accs.at[pl.ds(s * RPTA, RPTA)])
            plsc.subcore_barrier()

            fetch_idx(0, 0)
            drain_idx()
            pltpu.async_copy(ytbl.at[g].at[rring.at[0, 0]], gA, semA)
            pltpu.async_copy(ytbl.at[g].at[rring.at[0, 1]], gB, semB)

            @pl.loop(0, npair)
            def _(kk):
                cur = lax.rem(kk, 2)
                nxt = lax.rem(kk + 1, 2)
                not_last = kk < npair - 1

                @pl.when(not_last)
                def _():
                    fetch_idx(kk + 1, nxt)

                pltpu.make_async_copy(ytbl.at[g].at[rring.at[cur, 0]], gA,
                                      semA).wait()
                pltpu.sync_copy(gA, accs.at[cring.at[cur, 0]], add=True)

                @pl.when(not_last)
                def _():
                    drain_idx()
                    pltpu.async_copy(ytbl.at[g].at[rring.at[nxt, 0]], gA, semA)

                pltpu.make_async_copy(ytbl.at[g].at[rring.at[cur, 1]], gB,
                                      semB).wait()
                pltpu.sync_copy(gB, accs.at[cring.at[cur, 1]], add=True)

                @pl.when(not_last)
                def _():
                    pltpu.async_copy(ytbl.at[g].at[rring.at[nxt, 1]], gB, semB)

            plsc.subcore_barrier()
            pltpu.sync_copy(accs.at[pl.ds(s * RPTA, RPTA)],
                            out.at[p, pl.ds(s * RPTA, RPTA)])
            plsc.subcore_barrier()

    return prop


@functools.partial(
    pl.kernel,
    out_type=jax.ShapeDtypeStruct((NCORE, N_PAD, 128), jnp.float32),
    mesh=_mesh,
    scratch_types=[
        pltpu.VMEM((NCHK, S), jnp.int32),
        pltpu.VMEM((S, 128), jnp.float32),
        pltpu.VMEM_SHARED((N_PAD, 128), jnp.float32),
    ],
)
def _deg_kernel(colt, ones, zeros, out, col_v, ones_v, accs):
    ci = lax.axis_index("c")
    s = lax.axis_index("s")
    pltpu.sync_copy(colt.at[s], col_v)
    pltpu.sync_copy(ones, ones_v)
    pltpu.sync_copy(zeros, accs.at[pl.ds(s * RPT, RPT)])
    plsc.subcore_barrier()

    @pl.loop(0, NCHK // NCORE)
    def _(k):
        pltpu.sync_copy(ones_v, accs.at[col_v.at[ci * (NCHK // NCORE) + k]],
                        add=True)

    plsc.subcore_barrier()
    pltpu.sync_copy(accs.at[pl.ds(s * RPT, RPT)],
                    out.at[ci, pl.ds(s * RPT, RPT)])


@functools.partial(
    pl.kernel,
    out_type=[
        jax.ShapeDtypeStruct((NSUB, 2, EPTC), jnp.int32),
        jax.ShapeDtypeStruct((NSUB, 2, EPTC), jnp.int32),
        jax.ShapeDtypeStruct((NSUB, 16), jnp.int32),
    ],
    mesh=_mesh,
    scratch_types=[
        pltpu.VMEM((NCHK, S), jnp.int32),
        pltpu.VMEM((NCHK, S), jnp.int32),
        [pltpu.VMEM((EPTC,), jnp.int32) for _ in range(2)],
        [pltpu.VMEM((EPTC,), jnp.int32) for _ in range(2)],
        pltpu.VMEM((16,), jnp.int32),
    ],
    compiler_params=pltpu.CompilerParams(needs_layout_passes=False),
)
def _part_kernel(rowt, colt, rows_out, cols_out, npt_out, row_v, col_v,
                 prow, pcol, npv):
    ci = lax.axis_index("c")
    s = lax.axis_index("s")
    pltpu.sync_copy(rowt.at[s], row_v)
    pltpu.sync_copy(colt.at[s], col_v)

    def body(i, offs):
        off0, off1 = offs
        r = i // (S // 16)
        c = lax.rem(i, S // 16)
        rv = row_v[r, pl.ds(c * 16, 16)]
        cv = col_v[r, pl.ds(c * 16, 16)]
        m1 = cv >= NH
        m0 = jnp.logical_not(m1)
        n1 = plsc.all_reduce_population_count(m1)[0]
        plsc.store_compressed(prow[0].at[pl.ds(off0, 16)], rv, mask=m0)
        plsc.store_compressed(pcol[0].at[pl.ds(off0, 16)], cv, mask=m0)
        plsc.store_compressed(prow[1].at[pl.ds(off1, 16)], rv, mask=m1)
        plsc.store_compressed(pcol[1].at[pl.ds(off1, 16)], cv - NH, mask=m1)
        return off0 + (16 - n1), off1 + n1

    zero = jnp.zeros((), jnp.int32)
    cnt0, cnt1 = pl.loop(0, EPT // 16, init_carry=(zero, zero))(body)

    dumpv = jnp.full((16,), DUMPL, jnp.int32)
    zerov = jnp.zeros((16,), jnp.int32)
    for half, cnt in ((0, cnt0), (1, cnt1)):
        for k in range(16):
            prow[half][pl.ds(cnt + 16 * k, 16)] = zerov
            pcol[half][pl.ds(cnt + 16 * k, 16)] = dumpv

    npair0 = jnp.maximum((cnt0 + 2 * SG - 1) // (2 * SG), 1)
    npair1 = jnp.maximum((cnt1 + 2 * SG - 1) // (2 * SG), 1)
    lane = lax.iota(jnp.int32, 16)
    npv[...] = jnp.where(lane == 0, npair0,
                         jnp.where(lane == 1, npair1, 0))
    @pl.when(ci == 0)
    def _():
        for half in range(2):
            pltpu.sync_copy(prow[half], rows_out.at[s, half])
            pltpu.sync_copy(pcol[half], cols_out.at[s, half])
        pltpu.sync_copy(npv, npt_out.at[s])


def _dinv_of(degp_ref):
    return lax.rsqrt(degp_ref[0] + degp_ref[1] + 1.0)


def _tc_a_body(x_ref, h_ref, w_ref, degp_ref, y1_ref):
    dinv = _dinv_of(degp_ref)
    y = (jnp.dot(x_ref[0], w_ref[0:C], preferred_element_type=jnp.float32)
         + jnp.dot(h_ref[0], w_ref[C:C + H],
                   preferred_element_type=jnp.float32))
    y1_ref[0] = y * jnp.concatenate([dinv, dinv], axis=1)


def _tc_b_body(acc1_ref, y1_ref, degp_ref, x_ref, h_ref, wh_ref, bzr_ref,
               y2_ref, z_ref):
    for b in range(2):
        dinv = _dinv_of(degp_ref)
        z = jax.nn.sigmoid(dinv * (acc1_ref[b, 0, :, 0] + y1_ref[b][:, :H])
                           + bzr_ref[0])
        r = jax.nn.sigmoid(dinv * (acc1_ref[b, 0, :, 1] + y1_ref[b][:, H:])
                           + bzr_ref[1])
        rh = r * h_ref[b]
        y2 = (jnp.dot(x_ref[b], wh_ref[0:C],
                      preferred_element_type=jnp.float32)
              + jnp.dot(rh, wh_ref[C:C + H],
                        preferred_element_type=jnp.float32))
        y2_ref[0, :, b * H:(b + 1) * H] = y2 * dinv
        z_ref[b] = z


def _tc_c_body(acc2_ref, y2_ref, degp_ref, z_ref, h_ref, bh_ref, out_ref):
    for b in range(2):
        dinv = _dinv_of(degp_ref)
        ht = jnp.tanh(
            dinv * (acc2_ref[0, 0, :, b] + y2_ref[0, :, b * H:(b + 1) * H])
            + bh_ref[0])
        z = z_ref[b]
        out_ref[b] = (1.0 - z) * h_ref[b] + z * ht


def kernel(x, h, edge_index, W_zr, b_zr, W_h, b_h):
    row0 = edge_index[0]
    col0 = edge_index[1]
    colseg = col0.reshape(NSUB, EPT)
    row_d = jnp.pad(row0.reshape(NSUB, EPT),
                    ((0, 0), (0, EPT_PAD - EPT))).reshape(NSUB, NCHK, S)
    col_d = jnp.pad(colseg, ((0, 0), (0, EPT_PAD - EPT)),
                    constant_values=DUMP).reshape(NSUB, NCHK, S)

    rows_l, cols_l, npt = _part_kernel(row_d, col_d)
    rowp = rows_l.reshape(NSUB, 2, NCHKH, SG)
    colp = cols_l.reshape(NSUB, 2, NCHKH, SG)
    zeros_a = jnp.zeros((RPTA, 2, 128), jnp.float32)
    zeros_d = jnp.zeros((RPT, 128), jnp.float32)
    ones = jnp.ones((S, 128), jnp.float32)

    degp = _deg_kernel(col_d, ones, zeros_d)

    bnh = pl.BlockSpec((1, BLK, 128), lambda b, i: (b, i, 0))
    degp_b = pl.BlockSpec((NCORE, BLK, 128), lambda b, i: (0, i, 0))
    y1 = pl.pallas_call(
        _tc_a_body,
        grid=(B, N // BLK),
        in_specs=[
            bnh, bnh,
            pl.BlockSpec((C + H, 2 * H), lambda b, i: (0, 0)),
            degp_b,
        ],
        out_specs=pl.BlockSpec((1, BLK, 256), lambda b, i: (b, i, 0)),
        out_shape=jax.ShapeDtypeStruct((B, N, 256), jnp.float32),
    )(x, h, W_zr, degp)

    acc1 = _make_prop(2 * B)(y1.reshape(B, N, 2, 128), rowp, colp, npt,
                             zeros_a)
    acc1 = acc1.reshape(B, 2, NA_PAD, 2, 128)

    bnh2 = pl.BlockSpec((2, BLK, 128), lambda q, i: (q, i, 0))
    degp_b2 = pl.BlockSpec((NCORE, BLK, 128), lambda q, i: (0, i, 0))
    acc_b = pl.BlockSpec((2, 1, BLK, 2, 128),
                         lambda q, i: (q, i // NBH, i % NBH, 0, 0))
    y2, z = pl.pallas_call(
        _tc_b_body,
        grid=(B // 2, N // BLK),
        in_specs=[
            acc_b,
            pl.BlockSpec((2, BLK, 256), lambda q, i: (q, i, 0)),
            degp_b2,
            bnh2, bnh2,
            pl.BlockSpec((C + H, H), lambda q, i: (0, 0)),
            pl.BlockSpec((2, H), lambda q, i: (0, 0)),
        ],
        out_specs=[
            pl.BlockSpec((1, BLK, 256), lambda q, i: (q, i, 0)),
            bnh2,
        ],
        out_shape=[
            jax.ShapeDtypeStruct((B // 2, N, 256), jnp.float32),
            jax.ShapeDtypeStruct((B, N, 128), jnp.float32),
        ],
    )(acc1, y1, degp, x, h, W_h, b_zr.reshape(2, H))

    acc2 = _make_prop(B)(y2.reshape(B // 2, N, 2, 128), rowp, colp, npt,
                         zeros_a)
    acc2 = acc2.reshape(B // 2, 2, NA_PAD, 2, 128)

    out = pl.pallas_call(
        _tc_c_body,
        grid=(B // 2, N // BLK),
        in_specs=[
            pl.BlockSpec((1, 1, BLK, 2, 128),
                         lambda q, i: (q, i // NBH, i % NBH, 0, 0)),
            pl.BlockSpec((1, BLK, 256), lambda q, i: (q, i, 0)),
            degp_b2, bnh2, bnh2,
            pl.BlockSpec((1, H), lambda q, i: (0, 0)),
        ],
        out_specs=bnh2,
        out_shape=jax.ShapeDtypeStruct((B, N, H), jnp.float32),
    )(acc2, y2, degp, z, h, b_h.reshape(1, H))

    return out
